# baseline (speedup 1.0000x reference)
import functools

import jax
import jax.numpy as jnp
from jax import lax
from jax.experimental import pallas as pl
from jax.experimental.pallas import tpu as pltpu
from jax.experimental.pallas import tpu_sc as plsc

N = 10000
E = 320000
D = 128
DE = 16
LP0 = 64
NCLS = 2

_info = plsc.get_sparse_core_info()
NC = _info.num_cores
NS = _info.num_subcores
NW = NC * NS
EPW = E // NW
CHA = 80
NCHA = EPW // CHA
CHE = 80
NCHE = EPW // CHE
N_PAD = 10240
RPT = N_PAD // NS
L = 16
R = 4


def _leaky(v):
    return jnp.where(v >= 0, v, 0.2 * v)


_TE = 2560


def _ea_body(ea_ref, w0_ref, b0_ref, w1_ref, b1_ref, o0_ref, o1_ref):
    a = ea_ref[...]
    o0_ref[...] = jnp.dot(a, w0_ref[...], preferred_element_type=jnp.float32) + b0_ref[...]
    o1_ref[...] = jnp.dot(a, w1_ref[...], preferred_element_type=jnp.float32) + b1_ref[...]


def _ea_proj(edge_attr, w0, b0, w1, b1):
    return pl.pallas_call(
        _ea_body,
        grid=(E // _TE,),
        in_specs=[
            pl.BlockSpec((_TE, DE), lambda i: (i, 0)),
            pl.BlockSpec((DE, D), lambda i: (0, 0)),
            pl.BlockSpec((1, D), lambda i: (0, 0)),
            pl.BlockSpec((DE, D), lambda i: (0, 0)),
            pl.BlockSpec((1, D), lambda i: (0, 0)),
        ],
        out_specs=[
            pl.BlockSpec((_TE, D), lambda i: (i, 0)),
            pl.BlockSpec((_TE, D), lambda i: (i, 0)),
        ],
        out_shape=[
            jax.ShapeDtypeStruct((E, D), jnp.float32),
            jax.ShapeDtypeStruct((E, D), jnp.float32),
        ],
    )(edge_attr, w0, b0.reshape(1, D), w1, b1.reshape(1, D))


_sc_mesh = plsc.VectorSubcoreMesh(core_axis_name="c", subcore_axis_name="s")


def _ew_rows(ref_a, ref_b, n_rows, op):
    def _row(r, _):
        for c in range(D // L):
            s = pl.ds(c * L, L)
            ref_a[r, s] = op(ref_a[r, s], ref_b[r, s])
        return 0
    lax.fori_loop(0, n_rows, _row, 0, unroll=4)


RD = 2


@functools.partial(
    pl.kernel,
    out_type=jax.ShapeDtypeStruct((NC, N_PAD, D), jnp.float32),
    mesh=_sc_mesh,
    scratch_types=(
        [pltpu.VMEM((CHA,), jnp.int32) for _ in range(2 * R)]
        + [pltpu.VMEM((CHA, D), jnp.float32) for _ in range(2 * RD)]
        + [pltpu.VMEM_SHARED((N_PAD, D), jnp.float32)]
        + [pltpu.SemaphoreType.DMA for _ in range(R + 2 * RD)]
    ),
)
def _sc_aggregate(h_hbm, ea_hbm, src_hbm, dst_hbm, out_hbm, *scr):
    srcv = scr[0:R]
    dstv = scr[R:2 * R]
    rows = scr[2 * R:2 * R + RD]
    eav = scr[2 * R + RD:2 * R + 2 * RD]
    acc = scr[2 * R + 2 * RD]
    base_s = 2 * R + 2 * RD + 1
    isem = scr[base_s:base_s + R]
    gsem = scr[base_s + R:base_s + R + RD]
    ssem = scr[base_s + R + RD:base_s + R + 2 * RD]

    cid = lax.axis_index("c")
    sid = lax.axis_index("s")
    wid = sid * NC + cid
    e0 = wid * EPW
    last = NCHA - 1

    def idx_copies(c, q):
        base = e0 + c * CHA
        return (pltpu.make_async_copy(src_hbm.at[pl.ds(base, CHA)], srcv[q], isem[q]),
                pltpu.make_async_copy(dst_hbm.at[pl.ds(base, CHA)], dstv[q], isem[q]))

    def gather_copies(c, q, b):
        base = e0 + c * CHA
        return (pltpu.make_async_copy(ea_hbm.at[pl.ds(base, CHA)], eav[b], gsem[b]),
                pltpu.make_async_copy(h_hbm.at[srcv[q]], rows[b], gsem[b]))

    def issue_idx(c, q):
        for cp in idx_copies(c, q):
            cp.start()

    def wait_idx(c, q):
        for cp in idx_copies(c, q):
            cp.wait()

    def issue_gather(c, q, b):
        for cp in gather_copies(c, q, b):
            cp.start()

    def wait_gather(c, q, b):
        for cp in gather_copies(c, q, b):
            cp.wait()

    def issue_scatter(q, b):
        pltpu.async_copy(rows[b], acc.at[dstv[q]], ssem[b], add=True)

    def wait_scatter(q, b):
        pltpu.make_async_copy(rows[b], acc.at[dstv[q]], ssem[b]).wait()

    def _zrow(r, _):
        for c in range(D // L):
            rows[0][r, pl.ds(c * L, L)] = jnp.zeros((L,), jnp.float32)
        return 0
    lax.fori_loop(0, CHA, _zrow, 0)
    for k in range(RPT // CHA):
        pltpu.sync_copy(rows[0], acc.at[pl.ds(sid * RPT + k * CHA, CHA)])
    plsc.subcore_barrier()

    def steps(c, q, b, do_w):
        if do_w:
            wait_scatter((q + 3) % R, 1 - b)
        nq1, nq3 = (q + 1) % R, (q + 3) % R

        def pf_gather():
            wait_idx(c + 1, nq1)
            issue_gather(c + 1, nq1, 1 - b)

        def pf_idx():
            issue_idx(c + 3, nq3)

        if isinstance(c, int):
            if c + 1 <= last:
                pf_gather()
        else:
            pl.when(c + 1 <= last)(pf_gather)
        wait_gather(c, q, b)

        _ew_rows(rows[b], eav[b], CHA,
                 lambda a, v: jnp.maximum(a + v, 0.0))
        issue_scatter(q, b)
        if isinstance(c, int):
            if c + 3 <= last:
                pf_idx()
        else:
            pl.when(c + 3 <= last)(pf_idx)

    issue_idx(0, 0)
    issue_idx(1, 1)
    issue_idx(2, 2)
    wait_idx(0, 0)
    issue_gather(0, 0, 0)

    for c in range(R):
        steps(c, c % R, c % RD, do_w=(c >= 1))

    def _quad(t, _):
        for j in range(R):
            steps(t * R + j, j, j % RD, do_w=True)
        return 0
    lax.fori_loop(1, NCHA // R, _quad, 0)

    for c in range(NCHA - NCHA % R, NCHA):
        steps(c, c % R, c % RD, do_w=True)
    wait_scatter(last % R, last % RD)

    plsc.subcore_barrier()
    for k in range(RPT // CHA):
        off = sid * RPT + k * CHA
        pltpu.sync_copy(acc.at[pl.ds(off, CHA)], rows[k % RD])
        pltpu.sync_copy(rows[k % RD], out_hbm.at[cid, pl.ds(off, CHA)])


_TN = 2000


def _mlp_body(final_relu, ap_ref, h_ref, w_ref, b_ref, eps_ref, o_ref):
    agg = ap_ref[0] + ap_ref[1]
    hv = h_ref[...]
    v = (jnp.dot(agg, w_ref[:D], preferred_element_type=jnp.float32)
         + (1.0 + eps_ref[0, 0]) * jnp.dot(hv, w_ref[D:], preferred_element_type=jnp.float32)
         + b_ref[...])
    o_ref[...] = jnp.maximum(v, 0.0) if final_relu else _leaky(v)


def _mlp(agg_p, h, mw, mb, eps, final_relu):
    return pl.pallas_call(
        functools.partial(_mlp_body, final_relu),
        grid=(N // _TN,),
        in_specs=[
            pl.BlockSpec((NC, _TN, D), lambda i: (0, i, 0)),
            pl.BlockSpec((_TN, D), lambda i: (i, 0)),
            pl.BlockSpec((2 * D, D), lambda i: (0, 0)),
            pl.BlockSpec((1, D), lambda i: (0, 0)),
            pl.BlockSpec((1, 1), lambda i: (0, 0)),
        ],
        out_specs=pl.BlockSpec((_TN, D), lambda i: (i, 0)),
        out_shape=jax.ShapeDtypeStruct((N, D), jnp.float32),
    )(agg_p, h, mw, mb.reshape(1, D), eps.reshape(1, 1))


@functools.partial(
    pl.kernel,
    out_type=jax.ShapeDtypeStruct((E, D), jnp.float32),
    mesh=_sc_mesh,
    scratch_types=(
        [pltpu.VMEM((CHE,), jnp.int32) for _ in range(2 * R)]
        + [pltpu.VMEM((CHE, D), jnp.float32) for _ in range(2 * R)]
        + [pltpu.SemaphoreType.DMA for _ in range(3 * R)]
    ),
)
def _sc_edge_mult(h_hbm, src_hbm, dst_hbm, out_hbm, *scr):
    srcv = scr[0:R]
    dstv = scr[R:2 * R]
    rows_s = scr[2 * R:3 * R]
    rows_d = scr[3 * R:4 * R]
    isem = scr[4 * R:5 * R]
    gsem = scr[5 * R:6 * R]
    ssem = scr[6 * R:7 * R]

    cid = lax.axis_index("c")
    sid = lax.axis_index("s")
    wid = sid * NC + cid
    e0 = wid * EPW
    last = NCHE - 1

    def idx_copies(c, s):
        base = e0 + c * CHE
        return (pltpu.make_async_copy(src_hbm.at[pl.ds(base, CHE)], srcv[s], isem[s]),
                pltpu.make_async_copy(dst_hbm.at[pl.ds(base, CHE)], dstv[s], isem[s]))

    def gather_copies(c, s):
        return (pltpu.make_async_copy(h_hbm.at[srcv[s]], rows_s[s], gsem[s]),
                pltpu.make_async_copy(h_hbm.at[dstv[s]], rows_d[s], gsem[s]))

    def store_copy(c, s):
        base = e0 + c * CHE
        return pltpu.make_async_copy(rows_s[s], out_hbm.at[pl.ds(base, CHE)], ssem[s])

    def issue_idx(c, s):
        for cp in idx_copies(c, s):
            cp.start()

    def wait_idx(c, s):
        for cp in idx_copies(c, s):
            cp.wait()

    def issue_gather(c, s):
        for cp in gather_copies(c, s):
            cp.start()

    def wait_gather(c, s):
        for cp in gather_copies(c, s):
            cp.wait()

    def steps(c, b, do_d):
        wait_gather(c, b)
        _ew_rows(rows_s[b], rows_d[b], CHE, lambda a, v: a * v)
        store_copy(c, b).start()
        if do_d:
            pltpu.make_async_copy(
                rows_s[(b + 3) % R],
                out_hbm.at[pl.ds(0, CHE)],
                ssem[(b + 3) % R]).wait()
        nb3 = (b + 3) % R
        nb2 = (b + 2) % R
        if isinstance(c, int):
            if c + 3 <= last:
                issue_idx(c + 3, nb3)
            if c + 2 <= last:
                wait_idx(c + 2, nb2)
                issue_gather(c + 2, nb2)
        else:
            @pl.when(c + 3 <= last)
            def _():
                issue_idx(c + 3, nb3)

            @pl.when(c + 2 <= last)
            def _():
                wait_idx(c + 2, nb2)
                issue_gather(c + 2, nb2)

    issue_idx(0, 0)
    issue_idx(1, 1)
    wait_idx(0, 0)
    issue_gather(0, 0)
    wait_idx(1, 1)
    issue_gather(1, 1)
    issue_idx(2, 2)

    for c in range(R):
        steps(c, c % R, do_d=(c >= 1))

    def _quad(q, _):
        for j in range(R):
            steps(q * R + j, j, do_d=True)
        return 0
    lax.fori_loop(1, NCHE // R, _quad, 0)

    for c in range(NCHE - NCHE % R, NCHE):
        steps(c, c % R, do_d=True)
    pltpu.make_async_copy(rows_s[last % R], out_hbm.at[pl.ds(0, CHE)],
                          ssem[last % R]).wait()


def _lp_body(he_ref, w0_ref, b0_ref, w1_ref, b1_ref, o_ref):
    z = _leaky(jnp.dot(he_ref[...], w0_ref[...], preferred_element_type=jnp.float32)
               + b0_ref[...])
    logits = jnp.dot(z, w1_ref[...], preferred_element_type=jnp.float32) + b1_ref[...]
    m = jnp.max(logits, axis=-1, keepdims=True)
    ex = jnp.exp(logits - m)
    o_ref[...] = ex / jnp.sum(ex, axis=-1, keepdims=True)


def _lp_head(he, w0, b0, w1, b1):
    return pl.pallas_call(
        _lp_body,
        grid=(E // _TE,),
        in_specs=[
            pl.BlockSpec((_TE, D), lambda i: (i, 0)),
            pl.BlockSpec((D, LP0), lambda i: (0, 0)),
            pl.BlockSpec((1, LP0), lambda i: (0, 0)),
            pl.BlockSpec((LP0, NCLS), lambda i: (0, 0)),
            pl.BlockSpec((1, NCLS), lambda i: (0, 0)),
        ],
        out_specs=pl.BlockSpec((_TE, NCLS), lambda i: (i, 0)),
        out_shape=jax.ShapeDtypeStruct((E, NCLS), jnp.float32),
    )(he, w0, b0.reshape(1, LP0), w1, b1.reshape(1, NCLS))


def kernel(x, edge_index, edge_attr, anetW0, anetb0, mlpW0, mlpb0, eps0,
           anetW1, anetb1, mlpW1, mlpb1, eps1, lpW0, lpb0, lpW1, lpb1):
    src = edge_index[0]
    dst = edge_index[1]
    ea0, ea1 = _ea_proj(edge_attr, anetW0, anetb0, anetW1, anetb1)

    agg_p = _sc_aggregate(x, ea0, src, dst)
    h1 = _mlp(agg_p, x, mlpW0, mlpb0, eps0, final_relu=False)

    agg_p1 = _sc_aggregate(h1, ea1, src, dst)
    h2 = _mlp(agg_p1, h1, mlpW1, mlpb1, eps1, final_relu=True)

    he = _sc_edge_mult(h2, src, dst)
    yhat = _lp_head(he, lpW0, lpb0, lpW1, lpb1)
    return (he, yhat)


# speedup vs baseline: 1.6716x; 1.6716x over previous
import functools

import jax
import jax.numpy as jnp
from jax import lax
from jax.experimental import pallas as pl
from jax.experimental.pallas import tpu as pltpu
from jax.experimental.pallas import tpu_sc as plsc

N = 10000
E = 320000
D = 128
DE = 16
LP0 = 64
NCLS = 2

_info = plsc.get_sparse_core_info()
NC = _info.num_cores
NS = _info.num_subcores
NW = NC * NS
EPW = E // NW
CHA = 80
NCHA = EPW // CHA
CHE = 80
NCHE = EPW // CHE
N_PAD = 10240
RPT = N_PAD // NS
L = 16
R = 4


def _leaky(v):
    return jnp.where(v >= 0, v, 0.2 * v)


_TE = 2560


def _ea_body(ea_ref, w0_ref, b0_ref, w1_ref, b1_ref, o0_ref, o1_ref):
    a = ea_ref[...]
    o0_ref[...] = jnp.dot(a, w0_ref[...], preferred_element_type=jnp.float32) + b0_ref[...]
    o1_ref[...] = jnp.dot(a, w1_ref[...], preferred_element_type=jnp.float32) + b1_ref[...]


def _ea_proj(edge_attr, w0, b0, w1, b1):
    return pl.pallas_call(
        _ea_body,
        grid=(E // _TE,),
        in_specs=[
            pl.BlockSpec((_TE, DE), lambda i: (i, 0)),
            pl.BlockSpec((DE, D), lambda i: (0, 0)),
            pl.BlockSpec((1, D), lambda i: (0, 0)),
            pl.BlockSpec((DE, D), lambda i: (0, 0)),
            pl.BlockSpec((1, D), lambda i: (0, 0)),
        ],
        out_specs=[
            pl.BlockSpec((_TE, D), lambda i: (i, 0)),
            pl.BlockSpec((_TE, D), lambda i: (i, 0)),
        ],
        out_shape=[
            jax.ShapeDtypeStruct((E, D), jnp.float32),
            jax.ShapeDtypeStruct((E, D), jnp.float32),
        ],
    )(edge_attr, w0, b0.reshape(1, D), w1, b1.reshape(1, D))


_sc_mesh = plsc.VectorSubcoreMesh(core_axis_name="c", subcore_axis_name="s")


def _ew_rows(ref_a, ref_b, n_rows, op):
    @plsc.parallel_loop(0, n_rows, step=1)
    def _row(r):
        for c in range(D // L):
            s = pl.ds(c * L, L)
            ref_a[r, s] = op(ref_a[r, s], ref_b[r, s])


RD = 2


@functools.partial(
    pl.kernel,
    out_type=jax.ShapeDtypeStruct((NC, N_PAD, D), jnp.float32),
    mesh=_sc_mesh,
    scratch_types=(
        [pltpu.VMEM((CHA,), jnp.int32) for _ in range(2 * R)]
        + [pltpu.VMEM((CHA, D), jnp.float32) for _ in range(2 * RD)]
        + [pltpu.VMEM_SHARED((N_PAD, D), jnp.float32)]
        + [pltpu.SemaphoreType.DMA for _ in range(R + 2 * RD)]
    ),
)
def _sc_aggregate(h_hbm, ea_hbm, src_hbm, dst_hbm, out_hbm, *scr):
    srcv = scr[0:R]
    dstv = scr[R:2 * R]
    rows = scr[2 * R:2 * R + RD]
    eav = scr[2 * R + RD:2 * R + 2 * RD]
    acc = scr[2 * R + 2 * RD]
    base_s = 2 * R + 2 * RD + 1
    isem = scr[base_s:base_s + R]
    gsem = scr[base_s + R:base_s + R + RD]
    ssem = scr[base_s + R + RD:base_s + R + 2 * RD]

    cid = lax.axis_index("c")
    sid = lax.axis_index("s")
    wid = sid * NC + cid
    e0 = wid * EPW
    last = NCHA - 1

    def idx_copies(c, q):
        base = e0 + c * CHA
        return (pltpu.make_async_copy(src_hbm.at[pl.ds(base, CHA)], srcv[q], isem[q]),
                pltpu.make_async_copy(dst_hbm.at[pl.ds(base, CHA)], dstv[q], isem[q]))

    def gather_copies(c, q, b):
        base = e0 + c * CHA
        return (pltpu.make_async_copy(ea_hbm.at[pl.ds(base, CHA)], eav[b], gsem[b]),
                pltpu.make_async_copy(h_hbm.at[srcv[q]], rows[b], gsem[b]))

    def issue_idx(c, q):
        for cp in idx_copies(c, q):
            cp.start()

    def wait_idx(c, q):
        for cp in idx_copies(c, q):
            cp.wait()

    def issue_gather(c, q, b):
        for cp in gather_copies(c, q, b):
            cp.start()

    def wait_gather(c, q, b):
        for cp in gather_copies(c, q, b):
            cp.wait()

    def issue_scatter(q, b):
        pltpu.async_copy(rows[b], acc.at[dstv[q]], ssem[b], add=True)

    def wait_scatter(q, b):
        pltpu.make_async_copy(rows[b], acc.at[dstv[q]], ssem[b]).wait()

    def _zrow(r, _):
        for c in range(D // L):
            rows[0][r, pl.ds(c * L, L)] = jnp.zeros((L,), jnp.float32)
        return 0
    lax.fori_loop(0, CHA, _zrow, 0)
    for k in range(RPT // CHA):
        pltpu.sync_copy(rows[0], acc.at[pl.ds(sid * RPT + k * CHA, CHA)])
    plsc.subcore_barrier()

    def steps(c, q, b, do_w):
        if do_w:
            wait_scatter((q + 3) % R, 1 - b)
        nq1, nq3 = (q + 1) % R, (q + 3) % R

        def pf_gather():
            wait_idx(c + 1, nq1)
            issue_gather(c + 1, nq1, 1 - b)

        def pf_idx():
            issue_idx(c + 3, nq3)

        if isinstance(c, int):
            if c + 1 <= last:
                pf_gather()
        else:
            pl.when(c + 1 <= last)(pf_gather)
        wait_gather(c, q, b)

        _ew_rows(rows[b], eav[b], CHA,
                 lambda a, v: jnp.maximum(a + v, 0.0))
        issue_scatter(q, b)
        if isinstance(c, int):
            if c + 3 <= last:
                pf_idx()
        else:
            pl.when(c + 3 <= last)(pf_idx)

    issue_idx(0, 0)
    issue_idx(1, 1)
    issue_idx(2, 2)
    wait_idx(0, 0)
    issue_gather(0, 0, 0)

    for c in range(R):
        steps(c, c % R, c % RD, do_w=(c >= 1))

    def _quad(t, _):
        for j in range(R):
            steps(t * R + j, j, j % RD, do_w=True)
        return 0
    lax.fori_loop(1, NCHA // R, _quad, 0)

    for c in range(NCHA - NCHA % R, NCHA):
        steps(c, c % R, c % RD, do_w=True)
    wait_scatter(last % R, last % RD)

    plsc.subcore_barrier()
    for k in range(RPT // CHA):
        off = sid * RPT + k * CHA
        pltpu.sync_copy(acc.at[pl.ds(off, CHA)], rows[k % RD])
        pltpu.sync_copy(rows[k % RD], out_hbm.at[cid, pl.ds(off, CHA)])


_TN = 2000


def _mlp_body(final_relu, ap_ref, h_ref, w_ref, b_ref, eps_ref, o_ref):
    agg = ap_ref[0] + ap_ref[1]
    hv = h_ref[...]
    v = (jnp.dot(agg, w_ref[:D], preferred_element_type=jnp.float32)
         + (1.0 + eps_ref[0, 0]) * jnp.dot(hv, w_ref[D:], preferred_element_type=jnp.float32)
         + b_ref[...])
    o_ref[...] = jnp.maximum(v, 0.0) if final_relu else _leaky(v)


def _mlp(agg_p, h, mw, mb, eps, final_relu):
    return pl.pallas_call(
        functools.partial(_mlp_body, final_relu),
        grid=(N // _TN,),
        in_specs=[
            pl.BlockSpec((NC, _TN, D), lambda i: (0, i, 0)),
            pl.BlockSpec((_TN, D), lambda i: (i, 0)),
            pl.BlockSpec((2 * D, D), lambda i: (0, 0)),
            pl.BlockSpec((1, D), lambda i: (0, 0)),
            pl.BlockSpec((1, 1), lambda i: (0, 0)),
        ],
        out_specs=pl.BlockSpec((_TN, D), lambda i: (i, 0)),
        out_shape=jax.ShapeDtypeStruct((N, D), jnp.float32),
    )(agg_p, h, mw, mb.reshape(1, D), eps.reshape(1, 1))


@functools.partial(
    pl.kernel,
    out_type=jax.ShapeDtypeStruct((E, D), jnp.float32),
    mesh=_sc_mesh,
    scratch_types=(
        [pltpu.VMEM((CHE,), jnp.int32) for _ in range(2 * R)]
        + [pltpu.VMEM((CHE, D), jnp.float32) for _ in range(2 * R)]
        + [pltpu.SemaphoreType.DMA for _ in range(3 * R)]
    ),
)
def _sc_edge_mult(h_hbm, src_hbm, dst_hbm, out_hbm, *scr):
    srcv = scr[0:R]
    dstv = scr[R:2 * R]
    rows_s = scr[2 * R:3 * R]
    rows_d = scr[3 * R:4 * R]
    isem = scr[4 * R:5 * R]
    gsem = scr[5 * R:6 * R]
    ssem = scr[6 * R:7 * R]

    cid = lax.axis_index("c")
    sid = lax.axis_index("s")
    wid = sid * NC + cid
    e0 = wid * EPW
    last = NCHE - 1

    def idx_copies(c, s):
        base = e0 + c * CHE
        return (pltpu.make_async_copy(src_hbm.at[pl.ds(base, CHE)], srcv[s], isem[s]),
                pltpu.make_async_copy(dst_hbm.at[pl.ds(base, CHE)], dstv[s], isem[s]))

    def gather_copies(c, s):
        return (pltpu.make_async_copy(h_hbm.at[srcv[s]], rows_s[s], gsem[s]),
                pltpu.make_async_copy(h_hbm.at[dstv[s]], rows_d[s], gsem[s]))

    def store_copy(c, s):
        base = e0 + c * CHE
        return pltpu.make_async_copy(rows_s[s], out_hbm.at[pl.ds(base, CHE)], ssem[s])

    def issue_idx(c, s):
        for cp in idx_copies(c, s):
            cp.start()

    def wait_idx(c, s):
        for cp in idx_copies(c, s):
            cp.wait()

    def issue_gather(c, s):
        for cp in gather_copies(c, s):
            cp.start()

    def wait_gather(c, s):
        for cp in gather_copies(c, s):
            cp.wait()

    def steps(c, b, do_d):
        wait_gather(c, b)
        _ew_rows(rows_s[b], rows_d[b], CHE, lambda a, v: a * v)
        store_copy(c, b).start()
        if do_d:
            pltpu.make_async_copy(
                rows_s[(b + 3) % R],
                out_hbm.at[pl.ds(0, CHE)],
                ssem[(b + 3) % R]).wait()
        nb3 = (b + 3) % R
        nb2 = (b + 2) % R
        if isinstance(c, int):
            if c + 3 <= last:
                issue_idx(c + 3, nb3)
            if c + 2 <= last:
                wait_idx(c + 2, nb2)
                issue_gather(c + 2, nb2)
        else:
            @pl.when(c + 3 <= last)
            def _():
                issue_idx(c + 3, nb3)

            @pl.when(c + 2 <= last)
            def _():
                wait_idx(c + 2, nb2)
                issue_gather(c + 2, nb2)

    issue_idx(0, 0)
    issue_idx(1, 1)
    wait_idx(0, 0)
    issue_gather(0, 0)
    wait_idx(1, 1)
    issue_gather(1, 1)
    issue_idx(2, 2)

    for c in range(R):
        steps(c, c % R, do_d=(c >= 1))

    def _quad(q, _):
        for j in range(R):
            steps(q * R + j, j, do_d=True)
        return 0
    lax.fori_loop(1, NCHE // R, _quad, 0)

    for c in range(NCHE - NCHE % R, NCHE):
        steps(c, c % R, do_d=True)
    pltpu.make_async_copy(rows_s[last % R], out_hbm.at[pl.ds(0, CHE)],
                          ssem[last % R]).wait()


def _lp_body(he_ref, w0_ref, b0_ref, w1_ref, b1_ref, o_ref):
    z = _leaky(jnp.dot(he_ref[...], w0_ref[...], preferred_element_type=jnp.float32)
               + b0_ref[...])
    logits = jnp.dot(z, w1_ref[...], preferred_element_type=jnp.float32) + b1_ref[...]
    m = jnp.max(logits, axis=-1, keepdims=True)
    ex = jnp.exp(logits - m)
    o_ref[...] = ex / jnp.sum(ex, axis=-1, keepdims=True)


def _lp_head(he, w0, b0, w1, b1):
    return pl.pallas_call(
        _lp_body,
        grid=(E // _TE,),
        in_specs=[
            pl.BlockSpec((_TE, D), lambda i: (i, 0)),
            pl.BlockSpec((D, LP0), lambda i: (0, 0)),
            pl.BlockSpec((1, LP0), lambda i: (0, 0)),
            pl.BlockSpec((LP0, NCLS), lambda i: (0, 0)),
            pl.BlockSpec((1, NCLS), lambda i: (0, 0)),
        ],
        out_specs=pl.BlockSpec((_TE, NCLS), lambda i: (i, 0)),
        out_shape=jax.ShapeDtypeStruct((E, NCLS), jnp.float32),
    )(he, w0, b0.reshape(1, LP0), w1, b1.reshape(1, NCLS))


def kernel(x, edge_index, edge_attr, anetW0, anetb0, mlpW0, mlpb0, eps0,
           anetW1, anetb1, mlpW1, mlpb1, eps1, lpW0, lpb0, lpW1, lpb1):
    src = edge_index[0]
    dst = edge_index[1]
    ea0, ea1 = _ea_proj(edge_attr, anetW0, anetb0, anetW1, anetb1)

    agg_p = _sc_aggregate(x, ea0, src, dst)
    h1 = _mlp(agg_p, x, mlpW0, mlpb0, eps0, final_relu=False)

    agg_p1 = _sc_aggregate(h1, ea1, src, dst)
    h2 = _mlp(agg_p1, h1, mlpW1, mlpb1, eps1, final_relu=True)

    he = _sc_edge_mult(h2, src, dst)
    yhat = _lp_head(he, lpW0, lpb0, lpW1, lpb1)
    return (he, yhat)


# speedup vs baseline: 1.6888x; 1.0103x over previous
import functools

import numpy as np

import jax
import jax.numpy as jnp
from jax import lax
from jax.experimental import pallas as pl
from jax.experimental.pallas import tpu as pltpu
from jax.experimental.pallas import tpu_sc as plsc

N = 10000
E = 320000
D = 128
DE = 16
LP0 = 64
NCLS = 2

_info = plsc.get_sparse_core_info()
NC = _info.num_cores
NS = _info.num_subcores
NW = NC * NS
EPW = E // NW
CHA = 80
NCHA = EPW // CHA
CHE = 80
NCHE = EPW // CHE
N_PAD = 10240
RPT = N_PAD // NS
L = 16
R = 4


def _leaky(v):
    return jnp.where(v >= 0, v, 0.2 * v)


_TE = 2560


_PERM = np.empty((D,), np.int32)
for _h in range(2):
    for _t in range(D // 32):
        for _k in range(16):
            _PERM[64 * _h + 16 * _t + _k] = 32 * _t + 16 * _h + _k

_M16 = -65536


def _pack_words(v):
    vb = v.astype(jnp.bfloat16).astype(jnp.float32)
    wi = jax.lax.bitcast_convert_type(vb, jnp.int32)
    n = v.shape[-1] // 4
    pk0 = jax.lax.shift_right_logical(wi[:, :n], 16) | (wi[:, n:2 * n] & _M16)
    pk1 = jax.lax.shift_right_logical(wi[:, 2 * n:3 * n], 16) | (wi[:, 3 * n:] & _M16)
    return jnp.concatenate([pk0, pk1], axis=1)


def _ea_body(ea_ref, w0_ref, b0_ref, w1_ref, b1_ref, o0_ref, o1_ref):
    a = ea_ref[...]
    o0_ref[...] = _pack_words(
        jnp.dot(a, w0_ref[...], preferred_element_type=jnp.float32) + b0_ref[...])
    o1_ref[...] = _pack_words(
        jnp.dot(a, w1_ref[...], preferred_element_type=jnp.float32) + b1_ref[...])


def _blockdiag2(w):
    z = jnp.zeros_like(w)
    return jnp.concatenate(
        [jnp.concatenate([w, z], axis=1), jnp.concatenate([z, w], axis=1)], axis=0)


def _ea_proj(edge_attr, w0, b0, w1, b1):
    ea2 = edge_attr.reshape(E // 2, 2 * DE)
    wp0, wp1 = w0[:, _PERM], w1[:, _PERM]
    bp0, bp1 = b0[_PERM], b1[_PERM]
    return pl.pallas_call(
        _ea_body,
        grid=(E // _TE,),
        in_specs=[
            pl.BlockSpec((_TE // 2, 2 * DE), lambda i: (i, 0)),
            pl.BlockSpec((2 * DE, 2 * D), lambda i: (0, 0)),
            pl.BlockSpec((1, 2 * D), lambda i: (0, 0)),
            pl.BlockSpec((2 * DE, 2 * D), lambda i: (0, 0)),
            pl.BlockSpec((1, 2 * D), lambda i: (0, 0)),
        ],
        out_specs=[
            pl.BlockSpec((_TE // 2, D), lambda i: (i, 0)),
            pl.BlockSpec((_TE // 2, D), lambda i: (i, 0)),
        ],
        out_shape=[
            jax.ShapeDtypeStruct((E // 2, D), jnp.int32),
            jax.ShapeDtypeStruct((E // 2, D), jnp.int32),
        ],
    )(ea2, _blockdiag2(wp0), jnp.concatenate([bp0, bp0]).reshape(1, 2 * D),
      _blockdiag2(wp1), jnp.concatenate([bp1, bp1]).reshape(1, 2 * D))


_sc_mesh = plsc.VectorSubcoreMesh(core_axis_name="c", subcore_axis_name="s")


def _ew_rows(ref_a, ref_b, n_rows, op):
    @plsc.parallel_loop(0, n_rows, step=1)
    def _row(r):
        for c in range(D // L):
            s = pl.ds(c * L, L)
            ref_a[r, s] = op(ref_a[r, s], ref_b[r, s])


RD = 2


@functools.partial(
    pl.kernel,
    out_type=jax.ShapeDtypeStruct((NC, N_PAD, D), jnp.float32),
    mesh=_sc_mesh,
    scratch_types=(
        [pltpu.VMEM((CHA,), jnp.int32) for _ in range(2 * R)]
        + [pltpu.VMEM((CHA, D), jnp.float32) for _ in range(RD)]
        + [pltpu.VMEM((CHA // 2, D), jnp.int32) for _ in range(RD)]
        + [pltpu.VMEM_SHARED((N_PAD, D), jnp.float32)]
        + [pltpu.SemaphoreType.DMA for _ in range(R + 2 * RD)]
    ),
)
def _sc_aggregate(h_hbm, ea_hbm, src_hbm, dst_hbm, out_hbm, *scr):
    srcv = scr[0:R]
    dstv = scr[R:2 * R]
    rows = scr[2 * R:2 * R + RD]
    eav = scr[2 * R + RD:2 * R + 2 * RD]
    acc = scr[2 * R + 2 * RD]
    base_s = 2 * R + 2 * RD + 1
    isem = scr[base_s:base_s + R]
    gsem = scr[base_s + R:base_s + R + RD]
    ssem = scr[base_s + R + RD:base_s + R + 2 * RD]

    cid = lax.axis_index("c")
    sid = lax.axis_index("s")
    wid = sid * NC + cid
    e0 = wid * EPW
    last = NCHA - 1

    def idx_copies(c, q):
        base = e0 + c * CHA
        return (pltpu.make_async_copy(src_hbm.at[pl.ds(base, CHA)], srcv[q], isem[q]),
                pltpu.make_async_copy(dst_hbm.at[pl.ds(base, CHA)], dstv[q], isem[q]))

    def gather_copies(c, q, b):
        pbase = wid * (EPW // 2) + c * (CHA // 2)
        return (pltpu.make_async_copy(ea_hbm.at[pl.ds(pbase, CHA // 2)], eav[b], gsem[b]),
                pltpu.make_async_copy(h_hbm.at[srcv[q]], rows[b], gsem[b]))

    def issue_idx(c, q):
        for cp in idx_copies(c, q):
            cp.start()

    def wait_idx(c, q):
        for cp in idx_copies(c, q):
            cp.wait()

    def issue_gather(c, q, b):
        for cp in gather_copies(c, q, b):
            cp.start()

    def wait_gather(c, q, b):
        for cp in gather_copies(c, q, b):
            cp.wait()

    def issue_scatter(q, b):
        pltpu.async_copy(rows[b], acc.at[dstv[q]], ssem[b], add=True)

    def wait_scatter(q, b):
        pltpu.make_async_copy(rows[b], acc.at[dstv[q]], ssem[b]).wait()

    def _zrow(r, _):
        for c in range(D // L):
            rows[0][r, pl.ds(c * L, L)] = jnp.zeros((L,), jnp.float32)
        return 0
    lax.fori_loop(0, CHA, _zrow, 0)
    for k in range(RPT // CHA):
        pltpu.sync_copy(rows[0], acc.at[pl.ds(sid * RPT + k * CHA, CHA)])
    plsc.subcore_barrier()

    def steps(c, q, b, do_w):
        if do_w:
            wait_scatter((q + 3) % R, 1 - b)
        nq1, nq3 = (q + 1) % R, (q + 3) % R

        def pf_gather():
            wait_idx(c + 1, nq1)
            issue_gather(c + 1, nq1, 1 - b)

        def pf_idx():
            issue_idx(c + 3, nq3)

        if isinstance(c, int):
            if c + 1 <= last:
                pf_gather()
        else:
            pl.when(c + 1 <= last)(pf_gather)
        wait_gather(c, q, b)

        @plsc.parallel_loop(0, CHA // 2, step=1)
        def _pair(p):
            for j in range(2):
                r = p * 2 + j
                for t in range(D // 32):
                    w = eav[b][p, pl.ds(j * (D // 2) + t * L, L)]
                    lo = jax.lax.bitcast_convert_type(w << 16, jnp.float32)
                    hi = jax.lax.bitcast_convert_type(w & _M16, jnp.float32)
                    s0 = pl.ds(t * 32, L)
                    s1 = pl.ds(t * 32 + 16, L)
                    rows[b][r, s0] = jnp.maximum(rows[b][r, s0] + lo, 0.0)
                    rows[b][r, s1] = jnp.maximum(rows[b][r, s1] + hi, 0.0)
        issue_scatter(q, b)
        if isinstance(c, int):
            if c + 3 <= last:
                pf_idx()
        else:
            pl.when(c + 3 <= last)(pf_idx)

    issue_idx(0, 0)
    issue_idx(1, 1)
    issue_idx(2, 2)
    wait_idx(0, 0)
    issue_gather(0, 0, 0)

    for c in range(R):
        steps(c, c % R, c % RD, do_w=(c >= 1))

    def _quad(t, _):
        for j in range(R):
            steps(t * R + j, j, j % RD, do_w=True)
        return 0
    lax.fori_loop(1, NCHA // R, _quad, 0)

    for c in range(NCHA - NCHA % R, NCHA):
        steps(c, c % R, c % RD, do_w=True)
    wait_scatter(last % R, last % RD)

    plsc.subcore_barrier()
    for k in range(RPT // CHA):
        off = sid * RPT + k * CHA
        pltpu.sync_copy(acc.at[pl.ds(off, CHA)], rows[k % RD])
        pltpu.sync_copy(rows[k % RD], out_hbm.at[cid, pl.ds(off, CHA)])


_TN = 2000


def _mlp_body(final_relu, ap_ref, h_ref, w_ref, b_ref, eps_ref, o_ref):
    agg = ap_ref[0] + ap_ref[1]
    hv = h_ref[...]
    v = (jnp.dot(agg, w_ref[:D], preferred_element_type=jnp.float32)
         + (1.0 + eps_ref[0, 0]) * jnp.dot(hv, w_ref[D:], preferred_element_type=jnp.float32)
         + b_ref[...])
    o_ref[...] = jnp.maximum(v, 0.0) if final_relu else _leaky(v)


def _mlp(agg_p, h, mw, mb, eps, final_relu):
    return pl.pallas_call(
        functools.partial(_mlp_body, final_relu),
        grid=(N // _TN,),
        in_specs=[
            pl.BlockSpec((NC, _TN, D), lambda i: (0, i, 0)),
            pl.BlockSpec((_TN, D), lambda i: (i, 0)),
            pl.BlockSpec((2 * D, D), lambda i: (0, 0)),
            pl.BlockSpec((1, D), lambda i: (0, 0)),
            pl.BlockSpec((1, 1), lambda i: (0, 0)),
        ],
        out_specs=pl.BlockSpec((_TN, D), lambda i: (i, 0)),
        out_shape=jax.ShapeDtypeStruct((N, D), jnp.float32),
    )(agg_p, h, mw, mb.reshape(1, D), eps.reshape(1, 1))


@functools.partial(
    pl.kernel,
    out_type=jax.ShapeDtypeStruct((E, D), jnp.float32),
    mesh=_sc_mesh,
    scratch_types=(
        [pltpu.VMEM((CHE,), jnp.int32) for _ in range(2 * R)]
        + [pltpu.VMEM((CHE, D), jnp.float32) for _ in range(2 * R)]
        + [pltpu.SemaphoreType.DMA for _ in range(3 * R)]
    ),
)
def _sc_edge_mult(h_hbm, src_hbm, dst_hbm, out_hbm, *scr):
    srcv = scr[0:R]
    dstv = scr[R:2 * R]
    rows_s = scr[2 * R:3 * R]
    rows_d = scr[3 * R:4 * R]
    isem = scr[4 * R:5 * R]
    gsem = scr[5 * R:6 * R]
    ssem = scr[6 * R:7 * R]

    cid = lax.axis_index("c")
    sid = lax.axis_index("s")
    wid = sid * NC + cid
    e0 = wid * EPW
    last = NCHE - 1

    def idx_copies(c, s):
        base = e0 + c * CHE
        return (pltpu.make_async_copy(src_hbm.at[pl.ds(base, CHE)], srcv[s], isem[s]),
                pltpu.make_async_copy(dst_hbm.at[pl.ds(base, CHE)], dstv[s], isem[s]))

    def gather_copies(c, s):
        return (pltpu.make_async_copy(h_hbm.at[srcv[s]], rows_s[s], gsem[s]),
                pltpu.make_async_copy(h_hbm.at[dstv[s]], rows_d[s], gsem[s]))

    def store_copy(c, s):
        base = e0 + c * CHE
        return pltpu.make_async_copy(rows_s[s], out_hbm.at[pl.ds(base, CHE)], ssem[s])

    def issue_idx(c, s):
        for cp in idx_copies(c, s):
            cp.start()

    def wait_idx(c, s):
        for cp in idx_copies(c, s):
            cp.wait()

    def issue_gather(c, s):
        for cp in gather_copies(c, s):
            cp.start()

    def wait_gather(c, s):
        for cp in gather_copies(c, s):
            cp.wait()

    def steps(c, b, do_d):
        wait_gather(c, b)
        _ew_rows(rows_s[b], rows_d[b], CHE, lambda a, v: a * v)
        store_copy(c, b).start()
        if do_d:
            pltpu.make_async_copy(
                rows_s[(b + 3) % R],
                out_hbm.at[pl.ds(0, CHE)],
                ssem[(b + 3) % R]).wait()
        nb3 = (b + 3) % R
        nb2 = (b + 2) % R
        if isinstance(c, int):
            if c + 3 <= last:
                issue_idx(c + 3, nb3)
            if c + 2 <= last:
                wait_idx(c + 2, nb2)
                issue_gather(c + 2, nb2)
        else:
            @pl.when(c + 3 <= last)
            def _():
                issue_idx(c + 3, nb3)

            @pl.when(c + 2 <= last)
            def _():
                wait_idx(c + 2, nb2)
                issue_gather(c + 2, nb2)

    issue_idx(0, 0)
    issue_idx(1, 1)
    wait_idx(0, 0)
    issue_gather(0, 0)
    wait_idx(1, 1)
    issue_gather(1, 1)
    issue_idx(2, 2)

    for c in range(R):
        steps(c, c % R, do_d=(c >= 1))

    def _quad(q, _):
        for j in range(R):
            steps(q * R + j, j, do_d=True)
        return 0
    lax.fori_loop(1, NCHE // R, _quad, 0)

    for c in range(NCHE - NCHE % R, NCHE):
        steps(c, c % R, do_d=True)
    pltpu.make_async_copy(rows_s[last % R], out_hbm.at[pl.ds(0, CHE)],
                          ssem[last % R]).wait()


def _lp_body(he_ref, w0_ref, b0_ref, w1_ref, b1_ref, o_ref):
    z = _leaky(jnp.dot(he_ref[...], w0_ref[...], preferred_element_type=jnp.float32)
               + b0_ref[...])
    logits = jnp.dot(z, w1_ref[...], preferred_element_type=jnp.float32) + b1_ref[...]
    m = jnp.max(logits, axis=-1, keepdims=True)
    ex = jnp.exp(logits - m)
    o_ref[...] = ex / jnp.sum(ex, axis=-1, keepdims=True)


def _lp_head(he, w0, b0, w1, b1):
    return pl.pallas_call(
        _lp_body,
        grid=(E // _TE,),
        in_specs=[
            pl.BlockSpec((_TE, D), lambda i: (i, 0)),
            pl.BlockSpec((D, LP0), lambda i: (0, 0)),
            pl.BlockSpec((1, LP0), lambda i: (0, 0)),
            pl.BlockSpec((LP0, NCLS), lambda i: (0, 0)),
            pl.BlockSpec((1, NCLS), lambda i: (0, 0)),
        ],
        out_specs=pl.BlockSpec((_TE, NCLS), lambda i: (i, 0)),
        out_shape=jax.ShapeDtypeStruct((E, NCLS), jnp.float32),
    )(he, w0, b0.reshape(1, LP0), w1, b1.reshape(1, NCLS))


def kernel(x, edge_index, edge_attr, anetW0, anetb0, mlpW0, mlpb0, eps0,
           anetW1, anetb1, mlpW1, mlpb1, eps1, lpW0, lpb0, lpW1, lpb1):
    src = edge_index[0]
    dst = edge_index[1]
    ea0, ea1 = _ea_proj(edge_attr, anetW0, anetb0, anetW1, anetb1)

    agg_p = _sc_aggregate(x, ea0, src, dst)
    h1 = _mlp(agg_p, x, mlpW0, mlpb0, eps0, final_relu=False)

    agg_p1 = _sc_aggregate(h1, ea1, src, dst)
    h2 = _mlp(agg_p1, h1, mlpW1, mlpb1, eps1, final_relu=True)

    he = _sc_edge_mult(h2, src, dst)
    yhat = _lp_head(he, lpW0, lpb0, lpW1, lpb1)
    return (he, yhat)


# speedup vs baseline: 1.6914x; 1.0015x over previous
import functools

import numpy as np

import jax
import jax.numpy as jnp
from jax import lax
from jax.experimental import pallas as pl
from jax.experimental.pallas import tpu as pltpu
from jax.experimental.pallas import tpu_sc as plsc

N = 10000
E = 320000
D = 128
DE = 16
LP0 = 64
NCLS = 2

_info = plsc.get_sparse_core_info()
NC = _info.num_cores
NS = _info.num_subcores
NW = NC * NS
EPW = E // NW
CHA = 80
NCHA = EPW // CHA
CHE = 80
NCHE = EPW // CHE
N_PAD = 10240
RPT = N_PAD // NS
L = 16
R = 4


def _leaky(v):
    return jnp.where(v >= 0, v, 0.2 * v)


_TE = 2560


_PERM = np.empty((D,), np.int32)
for _h in range(2):
    for _t in range(D // 32):
        for _k in range(16):
            _PERM[64 * _h + 16 * _t + _k] = 32 * _t + 16 * _h + _k

_M16 = -65536


def _pack_words(v):
    vb = v.astype(jnp.bfloat16).astype(jnp.float32)
    wi = jax.lax.bitcast_convert_type(vb, jnp.int32)
    n = v.shape[-1] // 4
    pk0 = jax.lax.shift_right_logical(wi[:, :n], 16) | (wi[:, n:2 * n] & _M16)
    pk1 = jax.lax.shift_right_logical(wi[:, 2 * n:3 * n], 16) | (wi[:, 3 * n:] & _M16)
    return jnp.concatenate([pk0, pk1], axis=1)


def _ea_body(ea_ref, w0_ref, b0_ref, w1_ref, b1_ref, o0_ref, o1_ref):
    a = ea_ref[...]
    o0_ref[...] = _pack_words(
        jnp.dot(a, w0_ref[...], preferred_element_type=jnp.float32) + b0_ref[...])
    o1_ref[...] = _pack_words(
        jnp.dot(a, w1_ref[...], preferred_element_type=jnp.float32) + b1_ref[...])


def _blockdiag2(w):
    z = jnp.zeros_like(w)
    return jnp.concatenate(
        [jnp.concatenate([w, z], axis=1), jnp.concatenate([z, w], axis=1)], axis=0)


def _ea_proj(edge_attr, w0, b0, w1, b1):
    ea2 = edge_attr.reshape(E // 2, 2 * DE)
    wp0, wp1 = w0[:, _PERM], w1[:, _PERM]
    bp0, bp1 = b0[_PERM], b1[_PERM]
    return pl.pallas_call(
        _ea_body,
        grid=(E // _TE,),
        in_specs=[
            pl.BlockSpec((_TE // 2, 2 * DE), lambda i: (i, 0)),
            pl.BlockSpec((2 * DE, 2 * D), lambda i: (0, 0)),
            pl.BlockSpec((1, 2 * D), lambda i: (0, 0)),
            pl.BlockSpec((2 * DE, 2 * D), lambda i: (0, 0)),
            pl.BlockSpec((1, 2 * D), lambda i: (0, 0)),
        ],
        out_specs=[
            pl.BlockSpec((_TE // 2, D), lambda i: (i, 0)),
            pl.BlockSpec((_TE // 2, D), lambda i: (i, 0)),
        ],
        out_shape=[
            jax.ShapeDtypeStruct((E // 2, D), jnp.int32),
            jax.ShapeDtypeStruct((E // 2, D), jnp.int32),
        ],
    )(ea2, _blockdiag2(wp0), jnp.concatenate([bp0, bp0]).reshape(1, 2 * D),
      _blockdiag2(wp1), jnp.concatenate([bp1, bp1]).reshape(1, 2 * D))


_sc_mesh = plsc.VectorSubcoreMesh(core_axis_name="c", subcore_axis_name="s")


def _ew_rows(ref_a, ref_b, n_rows, op):
    @plsc.parallel_loop(0, n_rows, step=1)
    def _row(r):
        for c in range(D // L):
            s = pl.ds(c * L, L)
            ref_a[r, s] = op(ref_a[r, s], ref_b[r, s])


RD = 2


@functools.partial(
    pl.kernel,
    out_type=jax.ShapeDtypeStruct((NC, N_PAD, D), jnp.float32),
    mesh=_sc_mesh,
    scratch_types=(
        [pltpu.VMEM((CHA,), jnp.int32) for _ in range(2 * R)]
        + [pltpu.VMEM((CHA, D), jnp.float32) for _ in range(RD)]
        + [pltpu.VMEM((CHA // 2, D), jnp.int32) for _ in range(RD)]
        + [pltpu.VMEM_SHARED((N_PAD, D), jnp.float32)]
        + [pltpu.SemaphoreType.DMA for _ in range(R + 2 * RD)]
    ),
)
def _sc_aggregate(h_hbm, ea_hbm, eix_hbm, out_hbm, *scr):
    srcv = scr[0:R]
    dstv = scr[R:2 * R]
    rows = scr[2 * R:2 * R + RD]
    eav = scr[2 * R + RD:2 * R + 2 * RD]
    acc = scr[2 * R + 2 * RD]
    base_s = 2 * R + 2 * RD + 1
    isem = scr[base_s:base_s + R]
    gsem = scr[base_s + R:base_s + R + RD]
    ssem = scr[base_s + R + RD:base_s + R + 2 * RD]

    cid = lax.axis_index("c")
    sid = lax.axis_index("s")
    wid = sid * NC + cid
    e0 = wid * EPW
    last = NCHA - 1

    def idx_copies(c, q):
        base = e0 + c * CHA
        return (pltpu.make_async_copy(eix_hbm.at[pl.ds(base, CHA)], srcv[q], isem[q]),
                pltpu.make_async_copy(eix_hbm.at[pl.ds(E + base, CHA)], dstv[q], isem[q]))

    def gather_copies(c, q, b):
        pbase = wid * (EPW // 2) + c * (CHA // 2)
        return (pltpu.make_async_copy(ea_hbm.at[pl.ds(pbase, CHA // 2)], eav[b], gsem[b]),
                pltpu.make_async_copy(h_hbm.at[srcv[q]], rows[b], gsem[b]))

    def issue_idx(c, q):
        for cp in idx_copies(c, q):
            cp.start()

    def wait_idx(c, q):
        for cp in idx_copies(c, q):
            cp.wait()

    def issue_gather(c, q, b):
        for cp in gather_copies(c, q, b):
            cp.start()

    def wait_gather(c, q, b):
        for cp in gather_copies(c, q, b):
            cp.wait()

    def issue_scatter(q, b):
        pltpu.async_copy(rows[b], acc.at[dstv[q]], ssem[b], add=True)

    def wait_scatter(q, b):
        pltpu.make_async_copy(rows[b], acc.at[dstv[q]], ssem[b]).wait()

    def _zrow(r, _):
        for c in range(D // L):
            rows[0][r, pl.ds(c * L, L)] = jnp.zeros((L,), jnp.float32)
        return 0
    lax.fori_loop(0, CHA, _zrow, 0)
    for k in range(RPT // CHA):
        pltpu.sync_copy(rows[0], acc.at[pl.ds(sid * RPT + k * CHA, CHA)])
    plsc.subcore_barrier()

    def steps(c, q, b, do_w):
        if do_w:
            wait_scatter((q + 3) % R, 1 - b)
        nq1, nq3 = (q + 1) % R, (q + 3) % R

        def pf_gather():
            wait_idx(c + 1, nq1)
            issue_gather(c + 1, nq1, 1 - b)

        def pf_idx():
            issue_idx(c + 3, nq3)

        if isinstance(c, int):
            if c + 1 <= last:
                pf_gather()
        else:
            pl.when(c + 1 <= last)(pf_gather)
        wait_gather(c, q, b)

        @plsc.parallel_loop(0, CHA // 2, step=1)
        def _pair(p):
            for j in range(2):
                r = p * 2 + j
                for t in range(D // 32):
                    w = eav[b][p, pl.ds(j * (D // 2) + t * L, L)]
                    lo = jax.lax.bitcast_convert_type(w << 16, jnp.float32)
                    hi = jax.lax.bitcast_convert_type(w & _M16, jnp.float32)
                    s0 = pl.ds(t * 32, L)
                    s1 = pl.ds(t * 32 + 16, L)
                    rows[b][r, s0] = jnp.maximum(rows[b][r, s0] + lo, 0.0)
                    rows[b][r, s1] = jnp.maximum(rows[b][r, s1] + hi, 0.0)
        issue_scatter(q, b)
        if isinstance(c, int):
            if c + 3 <= last:
                pf_idx()
        else:
            pl.when(c + 3 <= last)(pf_idx)

    issue_idx(0, 0)
    issue_idx(1, 1)
    issue_idx(2, 2)
    wait_idx(0, 0)
    issue_gather(0, 0, 0)

    for c in range(R):
        steps(c, c % R, c % RD, do_w=(c >= 1))

    def _quad(t, _):
        for j in range(R):
            steps(t * R + j, j, j % RD, do_w=True)
        return 0
    lax.fori_loop(1, NCHA // R, _quad, 0)

    for c in range(NCHA - NCHA % R, NCHA):
        steps(c, c % R, c % RD, do_w=True)
    wait_scatter(last % R, last % RD)

    plsc.subcore_barrier()
    for k in range(RPT // CHA):
        off = sid * RPT + k * CHA
        pltpu.sync_copy(acc.at[pl.ds(off, CHA)], rows[k % RD])
        pltpu.sync_copy(rows[k % RD], out_hbm.at[cid, pl.ds(off, CHA)])


_TN = 2000


def _mlp_body(final_relu, ap_ref, h_ref, w_ref, b_ref, eps_ref, o_ref):
    agg = ap_ref[0] + ap_ref[1]
    hv = h_ref[...]
    v = (jnp.dot(agg, w_ref[:D], preferred_element_type=jnp.float32)
         + (1.0 + eps_ref[0, 0]) * jnp.dot(hv, w_ref[D:], preferred_element_type=jnp.float32)
         + b_ref[...])
    o_ref[...] = jnp.maximum(v, 0.0) if final_relu else _leaky(v)


def _mlp(agg_p, h, mw, mb, eps, final_relu):
    return pl.pallas_call(
        functools.partial(_mlp_body, final_relu),
        grid=(N // _TN,),
        in_specs=[
            pl.BlockSpec((NC, _TN, D), lambda i: (0, i, 0)),
            pl.BlockSpec((_TN, D), lambda i: (i, 0)),
            pl.BlockSpec((2 * D, D), lambda i: (0, 0)),
            pl.BlockSpec((1, D), lambda i: (0, 0)),
            pl.BlockSpec((1, 1), lambda i: (0, 0)),
        ],
        out_specs=pl.BlockSpec((_TN, D), lambda i: (i, 0)),
        out_shape=jax.ShapeDtypeStruct((N, D), jnp.float32),
    )(agg_p, h, mw, mb.reshape(1, D), eps.reshape(1, 1))


@functools.partial(
    pl.kernel,
    out_type=jax.ShapeDtypeStruct((E, D), jnp.float32),
    mesh=_sc_mesh,
    scratch_types=(
        [pltpu.VMEM((CHE,), jnp.int32) for _ in range(2 * R)]
        + [pltpu.VMEM((CHE, D), jnp.float32) for _ in range(2 * R)]
        + [pltpu.SemaphoreType.DMA for _ in range(3 * R)]
    ),
)
def _sc_edge_mult(h_hbm, eix_hbm, out_hbm, *scr):
    srcv = scr[0:R]
    dstv = scr[R:2 * R]
    rows_s = scr[2 * R:3 * R]
    rows_d = scr[3 * R:4 * R]
    isem = scr[4 * R:5 * R]
    gsem = scr[5 * R:6 * R]
    ssem = scr[6 * R:7 * R]

    cid = lax.axis_index("c")
    sid = lax.axis_index("s")
    wid = sid * NC + cid
    e0 = wid * EPW
    last = NCHE - 1

    def idx_copies(c, s):
        base = e0 + c * CHE
        return (pltpu.make_async_copy(eix_hbm.at[pl.ds(base, CHE)], srcv[s], isem[s]),
                pltpu.make_async_copy(eix_hbm.at[pl.ds(E + base, CHE)], dstv[s], isem[s]))

    def gather_copies(c, s):
        return (pltpu.make_async_copy(h_hbm.at[srcv[s]], rows_s[s], gsem[s]),
                pltpu.make_async_copy(h_hbm.at[dstv[s]], rows_d[s], gsem[s]))

    def store_copy(c, s):
        base = e0 + c * CHE
        return pltpu.make_async_copy(rows_s[s], out_hbm.at[pl.ds(base, CHE)], ssem[s])

    def issue_idx(c, s):
        for cp in idx_copies(c, s):
            cp.start()

    def wait_idx(c, s):
        for cp in idx_copies(c, s):
            cp.wait()

    def issue_gather(c, s):
        for cp in gather_copies(c, s):
            cp.start()

    def wait_gather(c, s):
        for cp in gather_copies(c, s):
            cp.wait()

    def steps(c, b, do_d):
        wait_gather(c, b)
        _ew_rows(rows_s[b], rows_d[b], CHE, lambda a, v: a * v)
        store_copy(c, b).start()
        if do_d:
            pltpu.make_async_copy(
                rows_s[(b + 3) % R],
                out_hbm.at[pl.ds(0, CHE)],
                ssem[(b + 3) % R]).wait()
        nb3 = (b + 3) % R
        nb2 = (b + 2) % R
        if isinstance(c, int):
            if c + 3 <= last:
                issue_idx(c + 3, nb3)
            if c + 2 <= last:
                wait_idx(c + 2, nb2)
                issue_gather(c + 2, nb2)
        else:
            @pl.when(c + 3 <= last)
            def _():
                issue_idx(c + 3, nb3)

            @pl.when(c + 2 <= last)
            def _():
                wait_idx(c + 2, nb2)
                issue_gather(c + 2, nb2)

    issue_idx(0, 0)
    issue_idx(1, 1)
    wait_idx(0, 0)
    issue_gather(0, 0)
    wait_idx(1, 1)
    issue_gather(1, 1)
    issue_idx(2, 2)

    for c in range(R):
        steps(c, c % R, do_d=(c >= 1))

    def _quad(q, _):
        for j in range(R):
            steps(q * R + j, j, do_d=True)
        return 0
    lax.fori_loop(1, NCHE // R, _quad, 0)

    for c in range(NCHE - NCHE % R, NCHE):
        steps(c, c % R, do_d=True)
    pltpu.make_async_copy(rows_s[last % R], out_hbm.at[pl.ds(0, CHE)],
                          ssem[last % R]).wait()


def _lp_body(he_ref, w0_ref, b0_ref, w1_ref, b1_ref, o_ref):
    z = _leaky(jnp.dot(he_ref[...], w0_ref[...], preferred_element_type=jnp.float32)
               + b0_ref[...])
    logits = jnp.dot(z, w1_ref[...], preferred_element_type=jnp.float32) + b1_ref[...]
    m = jnp.max(logits, axis=-1, keepdims=True)
    ex = jnp.exp(logits - m)
    o_ref[...] = ex / jnp.sum(ex, axis=-1, keepdims=True)


def _lp_head(he, w0, b0, w1, b1):
    return pl.pallas_call(
        _lp_body,
        grid=(E // _TE,),
        in_specs=[
            pl.BlockSpec((_TE, D), lambda i: (i, 0)),
            pl.BlockSpec((D, LP0), lambda i: (0, 0)),
            pl.BlockSpec((1, LP0), lambda i: (0, 0)),
            pl.BlockSpec((LP0, NCLS), lambda i: (0, 0)),
            pl.BlockSpec((1, NCLS), lambda i: (0, 0)),
        ],
        out_specs=pl.BlockSpec((_TE, NCLS), lambda i: (i, 0)),
        out_shape=jax.ShapeDtypeStruct((E, NCLS), jnp.float32),
    )(he, w0, b0.reshape(1, LP0), w1, b1.reshape(1, NCLS))


def kernel(x, edge_index, edge_attr, anetW0, anetb0, mlpW0, mlpb0, eps0,
           anetW1, anetb1, mlpW1, mlpb1, eps1, lpW0, lpb0, lpW1, lpb1):
    eix = edge_index.reshape(2 * E)
    ea0, ea1 = _ea_proj(edge_attr, anetW0, anetb0, anetW1, anetb1)

    agg_p = _sc_aggregate(x, ea0, eix)
    h1 = _mlp(agg_p, x, mlpW0, mlpb0, eps0, final_relu=False)

    agg_p1 = _sc_aggregate(h1, ea1, eix)
    h2 = _mlp(agg_p1, h1, mlpW1, mlpb1, eps1, final_relu=True)

    he = _sc_edge_mult(h2, eix)
    yhat = _lp_head(he, lpW0, lpb0, lpW1, lpb1)
    return (he, yhat)
